# baseline (speedup 1.0000x reference)
import jax
import jax.numpy as jnp
from jax.experimental import pallas as pl
from jax.experimental.pallas import tpu as pltpu

B, S, DIN, D, HID, E, NCLS = 64, 196, 768, 64, 256, 4, 10
T = B * S
SB = 49
NBLK = S // SB
BT = SB * B


def _gelu(v):
    return 0.5 * v * (1.0 + jax.lax.erf(v * 0.7071067811865476))


def _fused_body(x_ref, pe_w1t_ref, pe_b1_ref, pe_w2_ref, pe_b2_ref,
                gate_w_ref, gate_b_ref, exp_w1_ref, exp_b1_ref,
                exp_w2_ref, exp_b2_ref, ln_g_ref, ln_b_ref,
                cls_w1_ref, cls_b1_ref, cls_w2_ref, cls_b2_ref,
                logits_ref, moe_ref, aux_ref,
                pooled_acc, imp_acc, cnt_acc):
    i = pl.program_id(0)

    @pl.when(i == 0)
    def _():
        pooled_acc[...] = jnp.zeros_like(pooled_acc)
        imp_acc[...] = jnp.zeros_like(imp_acc)
        cnt_acc[...] = jnp.zeros_like(cnt_acc)

    xb = x_ref[...].reshape(BT, DIN)
    h1 = _gelu(
        jax.lax.dot_general(xb, pe_w1t_ref[...],
                            (((1,), (1,)), ((), ())),
                            preferred_element_type=jnp.float32)
        + pe_b1_ref[...])
    tok = (jnp.dot(h1, pe_w2_ref[...], preferred_element_type=jnp.float32)
           + pe_b2_ref[...])

    glog = (jax.lax.dot_general(tok, gate_w_ref[...],
                                (((1,), (1,)), ((), ())),
                                preferred_element_type=jnp.float32)
            + gate_b_ref[...])
    m = jnp.max(glog, axis=-1, keepdims=True)
    p = jnp.exp(glog - m)
    probs = p / jnp.sum(p, axis=-1, keepdims=True)
    gval = jnp.max(probs, axis=-1, keepdims=True)
    idx = jnp.argmax(probs, axis=-1).reshape(BT, 1)
    eids = jax.lax.broadcasted_iota(jnp.int32, (BT, E), 1)
    oh = (idx == eids).astype(jnp.float32)

    imp_acc[...] += jnp.sum(probs, axis=0, keepdims=True)
    cnt_acc[...] += jnp.sum(oh, axis=0, keepdims=True)

    combine = oh * gval
    tok16 = tok.astype(jnp.bfloat16)
    moe = jnp.dot(combine, exp_b2_ref[...],
                  preferred_element_type=jnp.float32)
    for e in range(E):
        he = _gelu(
            jnp.dot(tok16, exp_w1_ref[e].astype(jnp.bfloat16),
                    preferred_element_type=jnp.float32)
            + exp_b1_ref[e][None, :])
        oe = jnp.dot(he.astype(jnp.bfloat16),
                     exp_w2_ref[e].astype(jnp.bfloat16),
                     preferred_element_type=jnp.float32)
        moe += combine[:, e:e + 1] * oe

    moe3 = moe.reshape(SB, B, D)
    moe_ref[...] = moe3
    pooled_acc[...] += jnp.sum(moe3, axis=0)

    @pl.when(i == NBLK - 1)
    def _():
        pooled = pooled_acc[...] / S
        mu = jnp.mean(pooled, axis=-1, keepdims=True)
        var = jnp.mean((pooled - mu) ** 2, axis=-1, keepdims=True)
        ln = ((pooled - mu) / jnp.sqrt(var + 1e-5) * ln_g_ref[...]
              + ln_b_ref[...])
        c = _gelu(jnp.dot(ln, cls_w1_ref[...],
                          preferred_element_type=jnp.float32)
                  + cls_b1_ref[...])
        logits_ref[...] = (jax.lax.dot_general(
            cls_w2_ref[...], c, (((1,), (1,)), ((), ())),
            preferred_element_type=jnp.float32) + cls_b2_ref[...])
        imp = imp_acc[...] / T
        load = cnt_acc[...] / T
        aux = E * jnp.sum(imp * load)
        aux_ref[...] = jnp.full((1, 128), aux, dtype=jnp.float32)


def kernel(x, pe_w1, pe_b1, pe_w2, pe_b2, gate_w, gate_b, exp_w1, exp_b1,
           exp_w2, exp_b2, ln_g, ln_b, cls_w1, cls_b1, cls_w2, cls_b2,
           interpret=False):
    xt = jnp.transpose(x, (1, 0, 2))

    full = lambda shape: pl.BlockSpec(shape, lambda i: (0,) * len(shape))
    logits, moe_s, aux = pl.pallas_call(
        _fused_body,
        grid=(NBLK,),
        in_specs=[
            pl.BlockSpec((SB, B, DIN), lambda i: (i, 0, 0)),
            full((D, DIN)),
            full((1, D)),
            full((D, D)),
            full((1, D)),
            full((E, D)),
            full((1, E)),
            full((E, D, HID)),
            full((E, HID)),
            full((E, HID, D)),
            full((E, D)),
            full((1, D)),
            full((1, D)),
            full((D, D)),
            full((1, D)),
            full((NCLS, D)),
            full((NCLS, 1)),
        ],
        out_specs=[
            pl.BlockSpec((NCLS, B), lambda i: (0, 0)),
            pl.BlockSpec((SB, B, D), lambda i: (i, 0, 0)),
            pl.BlockSpec((1, 128), lambda i: (0, 0)),
        ],
        out_shape=[
            jax.ShapeDtypeStruct((NCLS, B), jnp.float32),
            jax.ShapeDtypeStruct((S, B, D), jnp.float32),
            jax.ShapeDtypeStruct((1, 128), jnp.float32),
        ],
        scratch_shapes=[
            pltpu.VMEM((B, D), jnp.float32),
            pltpu.VMEM((1, E), jnp.float32),
            pltpu.VMEM((1, E), jnp.float32),
        ],
        interpret=interpret,
    )(xt, pe_w1.T, pe_b1.reshape(1, D), pe_w2, pe_b2.reshape(1, D),
      gate_w.T, gate_b.reshape(1, E),
      exp_w1, exp_b1, exp_w2, exp_b2,
      ln_g.reshape(1, D), ln_b.reshape(1, D), cls_w1,
      cls_b1.reshape(1, D), cls_w2.T, cls_b2.reshape(NCLS, 1))
    moe_out = jnp.transpose(moe_s, (1, 0, 2))
    return logits.T, moe_out, aux[0, 0]


# speedup vs baseline: 1.0021x; 1.0021x over previous
import jax
import jax.numpy as jnp
from jax.experimental import pallas as pl
from jax.experimental.pallas import tpu as pltpu

B, S, DIN, D, HID, E, NCLS = 64, 196, 768, 64, 256, 4, 10
T = B * S
SB = 28
NBLK = S // SB
BT = SB * B


def _gelu(v):
    return 0.5 * v * (1.0 + jax.lax.erf(v * 0.7071067811865476))


def _fused_body(x_ref, pe_w1t_ref, pe_b1_ref, pe_w2_ref, pe_b2_ref,
                gate_w_ref, gate_b_ref, exp_w1_ref, exp_b1_ref,
                exp_w2_ref, exp_b2_ref, ln_g_ref, ln_b_ref,
                cls_w1_ref, cls_b1_ref, cls_w2_ref, cls_b2_ref,
                logits_ref, moe_ref, aux_ref,
                pooled_acc, imp_acc, cnt_acc):
    i = pl.program_id(0)

    @pl.when(i == 0)
    def _():
        pooled_acc[...] = jnp.zeros_like(pooled_acc)
        imp_acc[...] = jnp.zeros_like(imp_acc)
        cnt_acc[...] = jnp.zeros_like(cnt_acc)

    xb = x_ref[...].reshape(BT, DIN)
    h1 = _gelu(
        jax.lax.dot_general(xb, pe_w1t_ref[...],
                            (((1,), (1,)), ((), ())),
                            preferred_element_type=jnp.float32)
        + pe_b1_ref[...])
    tok = (jnp.dot(h1, pe_w2_ref[...], preferred_element_type=jnp.float32)
           + pe_b2_ref[...])

    glog = (jax.lax.dot_general(tok, gate_w_ref[...],
                                (((1,), (1,)), ((), ())),
                                preferred_element_type=jnp.float32)
            + gate_b_ref[...])
    m = jnp.max(glog, axis=-1, keepdims=True)
    p = jnp.exp(glog - m)
    probs = p / jnp.sum(p, axis=-1, keepdims=True)
    gval = jnp.max(probs, axis=-1, keepdims=True)
    idx = jnp.argmax(probs, axis=-1).reshape(BT, 1)
    eids = jax.lax.broadcasted_iota(jnp.int32, (BT, E), 1)
    oh = (idx == eids).astype(jnp.float32)

    imp_acc[...] += jnp.sum(probs, axis=0, keepdims=True)
    cnt_acc[...] += jnp.sum(oh, axis=0, keepdims=True)

    combine = oh * gval
    tok16 = tok.astype(jnp.bfloat16)
    moe = jnp.dot(combine, exp_b2_ref[...],
                  preferred_element_type=jnp.float32)
    for e in range(E):
        he = _gelu(
            jnp.dot(tok16, exp_w1_ref[e].astype(jnp.bfloat16),
                    preferred_element_type=jnp.float32)
            + exp_b1_ref[e][None, :])
        oe = jnp.dot(he.astype(jnp.bfloat16),
                     exp_w2_ref[e].astype(jnp.bfloat16),
                     preferred_element_type=jnp.float32)
        moe += combine[:, e:e + 1] * oe

    moe3 = moe.reshape(SB, B, D)
    moe_ref[...] = moe3
    pooled_acc[...] += jnp.sum(moe3, axis=0)

    @pl.when(i == NBLK - 1)
    def _():
        pooled = pooled_acc[...] / S
        mu = jnp.mean(pooled, axis=-1, keepdims=True)
        var = jnp.mean((pooled - mu) ** 2, axis=-1, keepdims=True)
        ln = ((pooled - mu) / jnp.sqrt(var + 1e-5) * ln_g_ref[...]
              + ln_b_ref[...])
        c = _gelu(jnp.dot(ln, cls_w1_ref[...],
                          preferred_element_type=jnp.float32)
                  + cls_b1_ref[...])
        logits_ref[...] = (jax.lax.dot_general(
            cls_w2_ref[...], c, (((1,), (1,)), ((), ())),
            preferred_element_type=jnp.float32) + cls_b2_ref[...])
        imp = imp_acc[...] / T
        load = cnt_acc[...] / T
        aux = E * jnp.sum(imp * load)
        aux_ref[...] = jnp.full((1, 128), aux, dtype=jnp.float32)


def kernel(x, pe_w1, pe_b1, pe_w2, pe_b2, gate_w, gate_b, exp_w1, exp_b1,
           exp_w2, exp_b2, ln_g, ln_b, cls_w1, cls_b1, cls_w2, cls_b2,
           interpret=False):
    xt = jnp.transpose(x, (1, 0, 2))

    full = lambda shape: pl.BlockSpec(shape, lambda i: (0,) * len(shape))
    logits, moe_s, aux = pl.pallas_call(
        _fused_body,
        grid=(NBLK,),
        in_specs=[
            pl.BlockSpec((SB, B, DIN), lambda i: (i, 0, 0)),
            full((D, DIN)),
            full((1, D)),
            full((D, D)),
            full((1, D)),
            full((E, D)),
            full((1, E)),
            full((E, D, HID)),
            full((E, HID)),
            full((E, HID, D)),
            full((E, D)),
            full((1, D)),
            full((1, D)),
            full((D, D)),
            full((1, D)),
            full((NCLS, D)),
            full((NCLS, 1)),
        ],
        out_specs=[
            pl.BlockSpec((NCLS, B), lambda i: (0, 0)),
            pl.BlockSpec((SB, B, D), lambda i: (i, 0, 0)),
            pl.BlockSpec((1, 128), lambda i: (0, 0)),
        ],
        out_shape=[
            jax.ShapeDtypeStruct((NCLS, B), jnp.float32),
            jax.ShapeDtypeStruct((S, B, D), jnp.float32),
            jax.ShapeDtypeStruct((1, 128), jnp.float32),
        ],
        scratch_shapes=[
            pltpu.VMEM((B, D), jnp.float32),
            pltpu.VMEM((1, E), jnp.float32),
            pltpu.VMEM((1, E), jnp.float32),
        ],
        interpret=interpret,
    )(xt, pe_w1.T, pe_b1.reshape(1, D), pe_w2, pe_b2.reshape(1, D),
      gate_w.T, gate_b.reshape(1, E),
      exp_w1, exp_b1, exp_w2, exp_b2,
      ln_g.reshape(1, D), ln_b.reshape(1, D), cls_w1,
      cls_b1.reshape(1, D), cls_w2.T, cls_b2.reshape(NCLS, 1))
    moe_out = jnp.transpose(moe_s, (1, 0, 2))
    return logits.T, moe_out, aux[0, 0]
